# baseline (speedup 1.0000x reference)
import jax
import jax.numpy as jnp
from jax import lax
from jax.experimental import pallas as pl
from jax.experimental.pallas import tpu as pltpu
from jax.experimental.pallas import tpu_sc as plsc

VOCAB = 1000000
EMB_DIM = 64
BATCH = 16384
HIST = 50

NUM_CORES = 2
NUM_SUBCORES = 16
NUM_WORKERS = NUM_CORES * NUM_SUBCORES

STRIPE = 256
S_ELEMS = STRIPE * EMB_DIM
N_FULL_STRIPES = VOCAB // STRIPE
TAIL_COLS = VOCAB - N_FULL_STRIPES * STRIPE

S_PER_W = BATCH // NUM_WORKERS
S_CHUNK = 16
N_CHUNKS = S_PER_W // S_CHUNK


def _transpose_stripe(in_v, out_v, n_cols):
    d_iota = lax.iota(jnp.int32, 16)

    @plsc.parallel_loop(0, n_cols, 1, unroll=8)
    def _col_body(l):
        idx_l = jnp.full((16,), l, jnp.int32)
        vs = [
            plsc.load_gather(in_v, [d_iota + d0, idx_l])
            for d0 in range(0, EMB_DIM, 16)
        ]
        for i, v in enumerate(vs):
            out_v[pl.ds(l * EMB_DIM + i * 16, 16)] = v


def _transpose_body(
    table_t_hbm, out_hbm,
    in_a, in_b, out_a, out_b, tail_in_v, tail_out_v,
    sia, sib, soa, sob,
):
    wid = lax.axis_index("s") * NUM_CORES + lax.axis_index("c")
    nk = (N_FULL_STRIPES - 1 - wid) // NUM_WORKERS + 1

    def in_slice(s):
        return table_t_hbm.at[:, pl.ds(s * STRIPE, STRIPE)]

    def in_dst(buf):
        return buf.at[:, pl.ds(0, STRIPE)]

    def out_slice(s):
        return out_hbm.at[pl.ds(s * S_ELEMS, S_ELEMS)]

    pltpu.async_copy(in_slice(wid), in_dst(in_a), sia)

    def ring_body(k2, carry):
        k0 = 2 * k2
        k1 = k0 + 1
        s0 = wid + k0 * NUM_WORKERS
        s1 = wid + k1 * NUM_WORKERS

        @pl.when(k1 < nk)
        def _fire_b():
            pltpu.async_copy(in_slice(s1), in_dst(in_b), sib)

        pltpu.make_async_copy(in_slice(s0), in_dst(in_a), sia).wait()

        @pl.when(k0 >= 2)
        def _drain_a():
            pltpu.make_async_copy(
                out_a, out_slice(s0 - 2 * NUM_WORKERS), soa
            ).wait()

        _transpose_stripe(in_a, out_a, STRIPE)
        pltpu.async_copy(out_a, out_slice(s0), soa)

        @pl.when(k1 < nk)
        def _do_b():
            @pl.when(k0 + 2 < nk)
            def _prefetch_a():
                pltpu.async_copy(in_slice(s0 + 2 * NUM_WORKERS), in_dst(in_a), sia)

            pltpu.make_async_copy(in_slice(s1), in_dst(in_b), sib).wait()

            @pl.when(k1 >= 2)
            def _drain_b():
                pltpu.make_async_copy(
                    out_b, out_slice(s1 - 2 * NUM_WORKERS), sob
                ).wait()

            _transpose_stripe(in_b, out_b, STRIPE)
            pltpu.async_copy(out_b, out_slice(s1), sob)

        return carry

    lax.fori_loop(0, (nk + 1) // 2, ring_body, 0)

    ka_last = ((nk - 1) // 2) * 2
    kb_last = ((nk - 2) // 2) * 2 + 1
    pltpu.make_async_copy(out_a, out_slice(wid + ka_last * NUM_WORKERS), soa).wait()
    pltpu.make_async_copy(out_b, out_slice(wid + kb_last * NUM_WORKERS), sob).wait()

    @pl.when(wid == NUM_WORKERS - 1)
    def _tail():
        c0 = N_FULL_STRIPES * STRIPE
        pltpu.sync_copy(table_t_hbm.at[:, pl.ds(c0, TAIL_COLS)], tail_in_v)
        _transpose_stripe(tail_in_v, tail_out_v, TAIL_COLS)
        pltpu.sync_copy(
            tail_out_v, out_hbm.at[pl.ds(c0 * EMB_DIM, TAIL_COLS * EMB_DIM)]
        )


def _gather_body(idx_hbm, table_hbm, out_hbm, idx_v, rows_v, gsem):
    wid = lax.axis_index("s") * NUM_CORES + lax.axis_index("c")
    base_s = wid * S_PER_W

    def chunk_body(i, carry):
        s0 = base_s + i * S_CHUNK
        pltpu.sync_copy(idx_hbm.at[pl.ds(s0, S_CHUNK)], idx_v)
        copies = [
            pltpu.async_copy(
                table_hbm.at[idx_v.at[j]],
                rows_v.at[j],
                gsem,
            )
            for j in range(S_CHUNK)
        ]
        for cp in copies:
            cp.wait()
        pltpu.sync_copy(rows_v, out_hbm.at[pl.ds(s0, S_CHUNK)])
        return carry

    lax.fori_loop(0, N_CHUNKS, chunk_body, 0)


@jax.jit
def kernel(batch, table):
    mesh = plsc.VectorSubcoreMesh(core_axis_name="c", subcore_axis_name="s")
    table_flat = pl.kernel(
        _transpose_body,
        out_type=jax.ShapeDtypeStruct((VOCAB * EMB_DIM,), jnp.float32),
        mesh=mesh,
        compiler_params=pltpu.CompilerParams(needs_layout_passes=False),
        scratch_types=[
            pltpu.VMEM((EMB_DIM, STRIPE + 1), jnp.float32),
            pltpu.VMEM((EMB_DIM, STRIPE + 1), jnp.float32),
            pltpu.VMEM((S_ELEMS,), jnp.float32),
            pltpu.VMEM((S_ELEMS,), jnp.float32),
            pltpu.VMEM((EMB_DIM, TAIL_COLS), jnp.float32),
            pltpu.VMEM((TAIL_COLS * EMB_DIM,), jnp.float32),
            pltpu.SemaphoreType.DMA,
            pltpu.SemaphoreType.DMA,
            pltpu.SemaphoreType.DMA,
            pltpu.SemaphoreType.DMA,
        ],
    )(table.T)
    table_rm = table_flat.reshape(VOCAB, EMB_DIM)
    return pl.kernel(
        _gather_body,
        out_type=jax.ShapeDtypeStruct((BATCH, HIST, EMB_DIM), jnp.float32),
        mesh=mesh,
        compiler_params=pltpu.CompilerParams(use_tc_tiling_on_sc=False),
        scratch_types=[
            pltpu.VMEM((S_CHUNK, HIST), jnp.int32),
            pltpu.VMEM((S_CHUNK, HIST, EMB_DIM), jnp.float32),
            pltpu.SemaphoreType.DMA,
        ],
    )(batch.astype(jnp.int32), table_rm)


# speedup vs baseline: 1.3633x; 1.3633x over previous
import jax
import jax.numpy as jnp
from jax import lax
from jax.experimental import pallas as pl
from jax.experimental.pallas import tpu as pltpu
from jax.experimental.pallas import tpu_sc as plsc

VOCAB = 1000000
EMB_DIM = 64
BATCH = 16384
HIST = 50

NUM_CORES = 2
NUM_SUBCORES = 16
NUM_WORKERS = NUM_CORES * NUM_SUBCORES

S_PER_W = BATCH // NUM_WORKERS
S_CHUNK = 16
N_CHUNKS = S_PER_W // S_CHUNK


def _gather_body(idx_hbm, table_hbm, out_hbm, idx_v, rows_v, gsem):
    wid = lax.axis_index("s") * NUM_CORES + lax.axis_index("c")
    base_s = wid * S_PER_W

    def chunk_body(i, carry):
        s0 = base_s + i * S_CHUNK
        pltpu.sync_copy(idx_hbm.at[pl.ds(s0, S_CHUNK)], idx_v)
        copies = [
            pltpu.async_copy(
                table_hbm.at[idx_v.at[j]],
                rows_v.at[j],
                gsem,
            )
            for j in range(S_CHUNK)
        ]
        for cp in copies:
            cp.wait()
        pltpu.sync_copy(rows_v, out_hbm.at[pl.ds(s0, S_CHUNK)])
        return carry

    lax.fori_loop(0, N_CHUNKS, chunk_body, 0)


@jax.jit
def kernel(batch, table):
    mesh = plsc.VectorSubcoreMesh(core_axis_name="c", subcore_axis_name="s")
    return pl.kernel(
        _gather_body,
        out_type=jax.ShapeDtypeStruct((BATCH, HIST, EMB_DIM), jnp.float32),
        mesh=mesh,
        compiler_params=pltpu.CompilerParams(use_tc_tiling_on_sc=False),
        scratch_types=[
            pltpu.VMEM((S_CHUNK, HIST), jnp.int32),
            pltpu.VMEM((S_CHUNK, HIST, EMB_DIM), jnp.float32),
            pltpu.SemaphoreType.DMA,
        ],
    )(batch.astype(jnp.int32), table)


# speedup vs baseline: 1.3858x; 1.0165x over previous
import jax
import jax.numpy as jnp
from jax import lax
from jax.experimental import pallas as pl
from jax.experimental.pallas import tpu as pltpu
from jax.experimental.pallas import tpu_sc as plsc

VOCAB = 1000000
EMB_DIM = 64
BATCH = 16384
HIST = 50

NUM_CORES = 2
NUM_SUBCORES = 16
NUM_WORKERS = NUM_CORES * NUM_SUBCORES

S_PER_W = BATCH // NUM_WORKERS
S_CHUNK = 16
N_CHUNKS = S_PER_W // S_CHUNK


def _gather_body(
    idx_hbm, table_hbm, out_hbm,
    idx_a, idx_b, rows_a, rows_b, g_a, g_b, so_a, so_b,
):
    wid = lax.axis_index("s") * NUM_CORES + lax.axis_index("c")
    base_s = wid * S_PER_W

    def do_chunk(i, idx_v, rows_v, gsem, osem):
        s0 = base_s + i * S_CHUNK
        pltpu.sync_copy(idx_hbm.at[pl.ds(s0, S_CHUNK)], idx_v)

        @pl.when(i >= 2)
        def _drain_prev():
            pltpu.make_async_copy(
                rows_v, out_hbm.at[pl.ds(s0 - 2 * S_CHUNK, S_CHUNK)], osem
            ).wait()

        copies = [
            pltpu.async_copy(
                table_hbm.at[idx_v.at[j]],
                rows_v.at[j],
                gsem,
            )
            for j in range(S_CHUNK)
        ]
        for cp in copies:
            cp.wait()
        pltpu.async_copy(rows_v, out_hbm.at[pl.ds(s0, S_CHUNK)], osem)

    def ring_body(k2, carry):
        do_chunk(2 * k2, idx_a, rows_a, g_a, so_a)
        do_chunk(2 * k2 + 1, idx_b, rows_b, g_b, so_b)
        return carry

    lax.fori_loop(0, N_CHUNKS // 2, ring_body, 0)

    last = base_s + (N_CHUNKS - 2) * S_CHUNK
    pltpu.make_async_copy(rows_a, out_hbm.at[pl.ds(last, S_CHUNK)], so_a).wait()
    pltpu.make_async_copy(
        rows_b, out_hbm.at[pl.ds(last + S_CHUNK, S_CHUNK)], so_b
    ).wait()


@jax.jit
def kernel(batch, table):
    mesh = plsc.VectorSubcoreMesh(core_axis_name="c", subcore_axis_name="s")
    return pl.kernel(
        _gather_body,
        out_type=jax.ShapeDtypeStruct((BATCH, HIST, EMB_DIM), jnp.float32),
        mesh=mesh,
        compiler_params=pltpu.CompilerParams(use_tc_tiling_on_sc=False),
        scratch_types=[
            pltpu.VMEM((S_CHUNK, HIST), jnp.int32),
            pltpu.VMEM((S_CHUNK, HIST), jnp.int32),
            pltpu.VMEM((S_CHUNK, HIST, EMB_DIM), jnp.float32),
            pltpu.VMEM((S_CHUNK, HIST, EMB_DIM), jnp.float32),
            pltpu.SemaphoreType.DMA,
            pltpu.SemaphoreType.DMA,
            pltpu.SemaphoreType.DMA,
            pltpu.SemaphoreType.DMA,
        ],
    )(batch.astype(jnp.int32), table)
